# baseline (speedup 1.0000x reference)
import functools

import jax
import jax.numpy as jnp
from jax import lax
from jax.experimental import pallas as pl
from jax.experimental.pallas import tpu as pltpu
from jax.experimental.pallas import tpu_sc as plsc

NX, NY, NZ, C = 432, 496, 1, 64
HW = NX * NY
P_PER_B = 10000

NUM_CORES = 2
NUM_SUBCORES = 16
NW = NUM_CORES * NUM_SUBCORES
NG = NY // 16

_SC_PARAMS = pltpu.CompilerParams(
    needs_layout_passes=False, use_tc_tiling_on_sc=False
)


def _worker_id():
    cid = lax.axis_index("c")
    sid = lax.axis_index("s")
    return sid * NUM_CORES + cid


def _sc_mesh():
    return plsc.VectorSubcoreMesh(
        core_axis_name="c", subcore_axis_name="s", num_cores=NUM_CORES
    )


def _build_inv(inv_ref, idx, vals, per_w):
    @functools.partial(
        pl.kernel,
        mesh=_sc_mesh(),
        out_type=(),
        scratch_types=[
            pltpu.VMEM((per_w,), jnp.int32),
            pltpu.VMEM((per_w,), jnp.int32),
            pltpu.SemaphoreType.DMA,
        ],
    )
    def inv_kernel(idx_hbm, val_hbm, inv, idx_v, val_v, sem):
        w = _worker_id()
        off = w * per_w
        pltpu.sync_copy(idx_hbm.at[pl.ds(off, per_w)], idx_v)
        pltpu.sync_copy(val_hbm.at[pl.ds(off, per_w)], val_v)
        pltpu.async_copy(val_v, inv.at[idx_v], sem).wait()

    inv_kernel(idx, vals, inv_ref)


def _dense_build(inv, pf_ext, batch_size):
    nrows = batch_size * NX
    rows_per_w = nrows // NW
    canvas_rows = batch_size * C * NX

    @functools.partial(
        pl.kernel,
        mesh=_sc_mesh(),
        compiler_params=_SC_PARAMS,
        out_type=jax.ShapeDtypeStruct((canvas_rows, NY), jnp.float32),
        scratch_types=[
            pltpu.VMEM((C, NY), jnp.float32),
            pltpu.VMEM((C, NY), jnp.float32),
            pltpu.VMEM((C,), jnp.int32),
            pltpu.VMEM((C,), jnp.int32),
            pltpu.VMEM((NY + 16,), jnp.int32),
            pltpu.VMEM((NY + 16,), jnp.int32),
            pltpu.VMEM((NY + 16,), jnp.int32),
            pltpu.VMEM((NY,), jnp.int32),
            pltpu.VMEM((NY,), jnp.int32),
            pltpu.VMEM((16,), jnp.int32),
            pltpu.VMEM((16, C), jnp.float32),
            pltpu.SMEM((2,), jnp.int32),
            pltpu.SemaphoreType.DMA,
            pltpu.SemaphoreType.DMA,
            pltpu.SemaphoreType.DMA,
            pltpu.SemaphoreType.DMA,
            pltpu.SemaphoreType.DMA,
        ],
    )
    def dense_kernel(inv_hbm, pf_hbm, canvas, tile0, tile1, rix0, rix1,
                     pos0, pos1, pid_v, inv0_v, inv1_v, pidg, rows_v, cnt_s,
                     gsem, isem0, isem1, osem0, osem1):
        w = _worker_id()
        lane = lax.iota(jnp.int32, 16)
        zero16 = jnp.zeros((16,), jnp.float32)
        tiles = (tile0, tile1)
        rixs = (rix0, rix1)
        poss = (pos0, pos1)
        osems = (osem0, osem1)
        invs = (inv0_v, inv1_v)
        isems = (isem0, isem1)

        def inv_addr(i):
            r = w + i * NW
            return (r // NX) * HW + (r % NX) * NY

        for zb in range(2):
            @pl.loop(0, C)
            def _(c, _zb=zb):
                cvec = jnp.broadcast_to(c, (16,)).astype(jnp.int32)
                for k in range(NG):
                    plsc.store_scatter(tiles[_zb], [cvec, lane + k * 16],
                                       zero16)

        cnt_s[0] = 0
        cnt_s[1] = 0

        def process(i, buf):
            tile, rix_v, pos_v, osem = (tiles[buf], rixs[buf], poss[buf],
                                        osems[buf])
            inv_v, isem = invs[buf], isems[buf]
            r = w + i * NW
            b = r // NX
            x = r % NX

            pltpu.make_async_copy(
                inv_hbm.at[pl.ds(0, NY)], inv_v, isem
            ).wait()

            @pl.when(i >= 2)
            def _():
                pltpu.make_async_copy(
                    canvas.at[pl.ds(0, C)], tile, osem
                ).wait()

            prev_cnt = cnt_s[buf]

            @pl.loop(0, (prev_cnt + 15) // 16)
            def _(jj):
                posv = plsc.load_gather(pos_v, [jj * 16 + lane])
                lm = (jj * 16 + lane) < prev_cnt
                for c in range(C):
                    cvec = jnp.full((16,), c, jnp.int32)
                    plsc.store_scatter(tile, [cvec, posv], zero16, mask=lm)

            def compress(k, cnt):
                v = inv_v[pl.ds(k * 16, 16)]
                msk = v > 0
                mi = msk.astype(jnp.int32)
                slot = cnt + plsc.cumsum(mi) - mi
                plsc.store_scatter(pid_v, [slot], v, mask=msk)
                plsc.store_scatter(pos_v, [slot], lane + k * 16, mask=msk)
                npk = jnp.max(plsc.all_reduce_population_count(msk))
                return cnt + npk

            cnt = lax.fori_loop(0, NG, compress, jnp.int32(0), unroll=True)
            cnt_s[buf] = cnt

            @pl.when(i + 2 < rows_per_w)
            def _():
                pltpu.async_copy(
                    inv_hbm.at[pl.ds(inv_addr(i + 2), NY)], inv_v, isem
                )

            for q in range(C // 16):
                rix_v[pl.ds(q * 16, 16)] = (
                    (b * C * NX + x) + NX * (lane + q * 16)
                )

            @pl.loop(0, (cnt + 15) // 16)
            def _(jj):
                lm = (jj * 16 + lane) < cnt
                pidv = plsc.load_gather(pid_v, [jj * 16 + lane])
                pidv = jnp.where(lm, pidv, 0)
                posv = plsc.load_gather(pos_v, [jj * 16 + lane])
                pidg[pl.ds(0, 16)] = pidv
                pltpu.async_copy(pf_hbm.at[pidg], rows_v, gsem).wait()
                for c in range(C):
                    cvec = jnp.full((16,), c, jnp.int32)
                    val = plsc.load_gather(rows_v, [lane, cvec])
                    plsc.store_scatter(tile, [cvec, posv], val, mask=lm)

            pltpu.async_copy(tile, canvas.at[rix_v], osem)

        pltpu.async_copy(inv_hbm.at[pl.ds(inv_addr(0), NY)], inv0_v, isem0)
        pltpu.async_copy(inv_hbm.at[pl.ds(inv_addr(1), NY)], inv1_v, isem1)

        @pl.loop(0, rows_per_w, step=2)
        def _(i):
            process(i, 0)
            process(i + 1, 1)

        for buf in range(2):
            pltpu.make_async_copy(
                canvas.at[pl.ds(0, C)], tiles[buf], osems[buf]
            ).wait()

    return dense_kernel(inv, pf_ext)


def kernel(pillar_features, voxel_coords, mask):
    m = mask.shape[0]
    coords = voxel_coords[:m, :].astype(jnp.int32)
    pf = pillar_features[:m, :] * mask[:, None].astype(pillar_features.dtype)
    batch_size = m // P_PER_B

    pix = coords[:, 1] * NY + coords[:, 2] + coords[:, 3]
    iidx = coords[:, 0] * HW + pix
    ival = jnp.arange(1, m + 1, dtype=jnp.int32)

    per_w = ((m + NW - 1) // NW + 7) // 8 * 8
    pad = NW * per_w - m
    dump = batch_size * HW
    iidx = jnp.concatenate([iidx, jnp.full((pad,), dump, jnp.int32)])
    ival = jnp.concatenate([ival, jnp.zeros((pad,), jnp.int32)])

    inv0 = jnp.zeros((batch_size * HW + 8,), jnp.int32)
    inv_ref = jax.new_ref(inv0)
    _build_inv(inv_ref, iidx, ival, per_w)
    inv = inv_ref[...]

    pf_ext = jnp.concatenate([jnp.zeros((1, C), pf.dtype), pf], axis=0)

    canvas = _dense_build(inv, pf_ext, batch_size)
    return canvas.reshape(batch_size, C * NZ, NX, NY).swapaxes(2, 3)


# speedup vs baseline: 1.0102x; 1.0102x over previous
import functools

import jax
import jax.numpy as jnp
from jax import lax
from jax.experimental import pallas as pl
from jax.experimental.pallas import tpu as pltpu
from jax.experimental.pallas import tpu_sc as plsc

NX, NY, NZ, C = 432, 496, 1, 64
HW = NX * NY
P_PER_B = 10000

NUM_CORES = 2
NUM_SUBCORES = 16
NW = NUM_CORES * NUM_SUBCORES
NG = NY // 16

_SC_PARAMS = pltpu.CompilerParams(
    needs_layout_passes=False, use_tc_tiling_on_sc=False
)


def _worker_id():
    cid = lax.axis_index("c")
    sid = lax.axis_index("s")
    return sid * NUM_CORES + cid


def _sc_mesh():
    return plsc.VectorSubcoreMesh(
        core_axis_name="c", subcore_axis_name="s", num_cores=NUM_CORES
    )


def _build_inv(inv_ref, idx, vals, per_w):
    @functools.partial(
        pl.kernel,
        mesh=_sc_mesh(),
        out_type=(),
        scratch_types=[
            pltpu.VMEM((per_w,), jnp.int32),
            pltpu.VMEM((per_w,), jnp.int32),
            pltpu.SemaphoreType.DMA,
        ],
    )
    def inv_kernel(idx_hbm, val_hbm, inv, idx_v, val_v, sem):
        w = _worker_id()
        off = w * per_w
        pltpu.sync_copy(idx_hbm.at[pl.ds(off, per_w)], idx_v)
        pltpu.sync_copy(val_hbm.at[pl.ds(off, per_w)], val_v)
        pltpu.async_copy(val_v, inv.at[idx_v], sem).wait()

    inv_kernel(idx, vals, inv_ref)


def _dense_build(inv, pf_ext, batch_size):
    nrows = batch_size * NX
    rows_per_w = nrows // NW
    canvas_rows = batch_size * C * NX

    @functools.partial(
        pl.kernel,
        mesh=_sc_mesh(),
        compiler_params=_SC_PARAMS,
        out_type=jax.ShapeDtypeStruct((canvas_rows, NY), jnp.float32),
        scratch_types=[
            pltpu.VMEM((C, NY), jnp.float32),
            pltpu.VMEM((C, NY), jnp.float32),
            pltpu.VMEM((C,), jnp.int32),
            pltpu.VMEM((C,), jnp.int32),
            pltpu.VMEM((NY + 16,), jnp.int32),
            pltpu.VMEM((NY + 16,), jnp.int32),
            pltpu.VMEM((NY + 16,), jnp.int32),
            pltpu.VMEM((NY,), jnp.int32),
            pltpu.VMEM((NY,), jnp.int32),
            pltpu.VMEM((16,), jnp.int32),
            pltpu.VMEM((16,), jnp.int32),
            pltpu.VMEM((16, C), jnp.float32),
            pltpu.VMEM((16, C), jnp.float32),
            pltpu.SMEM((2,), jnp.int32),
            pltpu.SemaphoreType.DMA,
            pltpu.SemaphoreType.DMA,
            pltpu.SemaphoreType.DMA,
            pltpu.SemaphoreType.DMA,
            pltpu.SemaphoreType.DMA,
            pltpu.SemaphoreType.DMA,
        ],
    )
    def dense_kernel(inv_hbm, pf_hbm, canvas, tile0, tile1, rix0, rix1,
                     pos0, pos1, pid_v, inv0_v, inv1_v, pidg0, pidg1,
                     rows0_v, rows1_v, cnt_s, gsem0, gsem1, isem0, isem1,
                     osem0, osem1):
        w = _worker_id()
        lane = lax.iota(jnp.int32, 16)
        zero16 = jnp.zeros((16,), jnp.float32)
        tiles = (tile0, tile1)
        rixs = (rix0, rix1)
        poss = (pos0, pos1)
        osems = (osem0, osem1)
        invs = (inv0_v, inv1_v)
        isems = (isem0, isem1)
        pidgs = (pidg0, pidg1)
        rowss = (rows0_v, rows1_v)
        gsems = (gsem0, gsem1)

        def inv_addr(i):
            r = w + i * NW
            return (r // NX) * HW + (r % NX) * NY

        for zb in range(2):
            @pl.loop(0, C)
            def _(c, _zb=zb):
                cvec = jnp.broadcast_to(c, (16,)).astype(jnp.int32)
                for k in range(NG):
                    plsc.store_scatter(tiles[_zb], [cvec, lane + k * 16],
                                       zero16)

        cnt_s[0] = 0
        cnt_s[1] = 0

        def process(i, buf):
            tile, rix_v, pos_v, osem = (tiles[buf], rixs[buf], poss[buf],
                                        osems[buf])
            inv_v, isem = invs[buf], isems[buf]
            r = w + i * NW
            b = r // NX
            x = r % NX

            pltpu.make_async_copy(
                inv_hbm.at[pl.ds(0, NY)], inv_v, isem
            ).wait()

            @pl.when(i >= 2)
            def _():
                pltpu.make_async_copy(
                    canvas.at[pl.ds(0, C)], tile, osem
                ).wait()

            prev_cnt = cnt_s[buf]

            @pl.loop(0, (prev_cnt + 15) // 16)
            def _(jj):
                posv = plsc.load_gather(pos_v, [jj * 16 + lane])
                lm = (jj * 16 + lane) < prev_cnt
                for c in range(C):
                    cvec = jnp.full((16,), c, jnp.int32)
                    plsc.store_scatter(tile, [cvec, posv], zero16, mask=lm)

            def compress(k, cnt):
                v = inv_v[pl.ds(k * 16, 16)]
                msk = v > 0
                mi = msk.astype(jnp.int32)
                slot = cnt + plsc.cumsum(mi) - mi
                plsc.store_scatter(pid_v, [slot], v, mask=msk)
                plsc.store_scatter(pos_v, [slot], lane + k * 16, mask=msk)
                npk = jnp.max(plsc.all_reduce_population_count(msk))
                return cnt + npk

            cnt = lax.fori_loop(0, NG, compress, jnp.int32(0), unroll=True)
            cnt_s[buf] = cnt

            @pl.when(i + 2 < rows_per_w)
            def _():
                pltpu.async_copy(
                    inv_hbm.at[pl.ds(inv_addr(i + 2), NY)], inv_v, isem
                )

            for q in range(C // 16):
                rix_v[pl.ds(q * 16, 16)] = (
                    (b * C * NX + x) + NX * (lane + q * 16)
                )

            ngroups = (cnt + 15) // 16

            def stage_fire(jj, gb):
                lm = (jj * 16 + lane) < cnt
                pidv = plsc.load_gather(pid_v, [jj * 16 + lane])
                pidv = jnp.where(lm, pidv, 0)
                pidgs[gb][pl.ds(0, 16)] = pidv
                pltpu.async_copy(pf_hbm.at[pidgs[gb]], rowss[gb], gsems[gb])

            def patch(jj, gb):
                lm = (jj * 16 + lane) < cnt
                posv = plsc.load_gather(pos_v, [jj * 16 + lane])
                pltpu.make_async_copy(
                    pf_hbm.at[pidgs[gb]], rowss[gb], gsems[gb]
                ).wait()
                for c in range(C):
                    cvec = jnp.full((16,), c, jnp.int32)
                    val = plsc.load_gather(rowss[gb], [lane, cvec])
                    plsc.store_scatter(tile, [cvec, posv], val, mask=lm)

            @pl.when(ngroups > 0)
            def _():
                stage_fire(0, 0)

            @pl.loop(0, ngroups, step=2)
            def _(jj):
                @pl.when(jj + 1 < ngroups)
                def _():
                    stage_fire(jj + 1, 1)

                patch(jj, 0)

                @pl.when(jj + 2 < ngroups)
                def _():
                    stage_fire(jj + 2, 0)

                @pl.when(jj + 1 < ngroups)
                def _():
                    patch(jj + 1, 1)

            pltpu.async_copy(tile, canvas.at[rix_v], osem)

        pltpu.async_copy(inv_hbm.at[pl.ds(inv_addr(0), NY)], inv0_v, isem0)
        pltpu.async_copy(inv_hbm.at[pl.ds(inv_addr(1), NY)], inv1_v, isem1)

        @pl.loop(0, rows_per_w, step=2)
        def _(i):
            process(i, 0)
            process(i + 1, 1)

        for buf in range(2):
            pltpu.make_async_copy(
                canvas.at[pl.ds(0, C)], tiles[buf], osems[buf]
            ).wait()

    return dense_kernel(inv, pf_ext)


def kernel(pillar_features, voxel_coords, mask):
    m = mask.shape[0]
    coords = voxel_coords[:m, :].astype(jnp.int32)
    pf = pillar_features[:m, :] * mask[:, None].astype(pillar_features.dtype)
    batch_size = m // P_PER_B

    pix = coords[:, 1] * NY + coords[:, 2] + coords[:, 3]
    iidx = coords[:, 0] * HW + pix
    ival = jnp.arange(1, m + 1, dtype=jnp.int32)

    per_w = ((m + NW - 1) // NW + 7) // 8 * 8
    pad = NW * per_w - m
    dump = batch_size * HW
    iidx = jnp.concatenate([iidx, jnp.full((pad,), dump, jnp.int32)])
    ival = jnp.concatenate([ival, jnp.zeros((pad,), jnp.int32)])

    inv0 = jnp.zeros((batch_size * HW + 8,), jnp.int32)
    inv_ref = jax.new_ref(inv0)
    _build_inv(inv_ref, iidx, ival, per_w)
    inv = inv_ref[...]

    pf_ext = jnp.concatenate([jnp.zeros((1, C), pf.dtype), pf], axis=0)

    canvas = _dense_build(inv, pf_ext, batch_size)
    return canvas.reshape(batch_size, C * NZ, NX, NY).swapaxes(2, 3)
